# Initial kernel scaffold; baseline (speedup 1.0000x reference)
import jax
import jax.numpy as jnp
from jax.experimental import pallas as pl


def kernel(x_fp16, ei, batch, mask, W1, b1, W2, b2, Wout, bout):
    raise NotImplementedError("write your pallas kernel here")


# speedup vs baseline: 12.6802x; 12.6802x over previous
import functools

import jax
import jax.numpy as jnp
from jax import lax
from jax.experimental import pallas as pl
from jax.experimental.pallas import tpu as pltpu
from jax.experimental.pallas import tpu_sc as plsc

_N = 10000
_E = 160000
_H = 128
_G = 64
_EPS = 1e-6

_NC, _NS = 2, 16
_NW = _NC * _NS
_CHUNK = 128
_NCHUNKS = _E // _CHUNK
_NPAD = 10112
_RPS = _NPAD // _NS

def _chunks_for(w):
    base, rem = divmod(_NCHUNKS, _NW)
    return base + jnp.where(w < rem, 1, 0)


@functools.cache
def _sc_kernels():
    mesh = plsc.VectorSubcoreMesh(
        core_axis_name="c", subcore_axis_name="s",
        num_cores=_NC, num_subcores=_NS,
    )

    @functools.partial(
        pl.kernel,
        out_type=jax.ShapeDtypeStruct((_NC, _NPAD, _H), jnp.float32),
        mesh=mesh,
        scratch_types=[
            pltpu.VMEM((_CHUNK,), jnp.int32),
            pltpu.VMEM((_CHUNK, _H), jnp.float32),
            pltpu.VMEM_SHARED((_NPAD, _H), jnp.float32),
        ],
    )
    def deg_kernel(dst_hbm, ones_hbm, zeros_hbm, out_hbm, dst_v, ones_v, acc_sh):
        c = lax.axis_index("c")
        s = lax.axis_index("s")
        w = s * _NC + c
        sl = pl.ds(s * _RPS, _RPS)
        pltpu.sync_copy(zeros_hbm.at[sl], acc_sh.at[sl])
        pltpu.sync_copy(ones_hbm, ones_v)
        plsc.subcore_barrier()

        @pl.loop(0, _chunks_for(w))
        def _(t):
            base = (w + t * _NW) * _CHUNK
            pltpu.sync_copy(dst_hbm.at[pl.ds(base, _CHUNK)], dst_v)
            pltpu.sync_copy(ones_v, acc_sh.at[dst_v], add=True)

        plsc.subcore_barrier()
        pltpu.sync_copy(acc_sh.at[sl], out_hbm.at[c, sl])

    @functools.partial(
        pl.kernel,
        out_type=jax.ShapeDtypeStruct((_NC, _NPAD, _H), jnp.float32),
        mesh=mesh,
        scratch_types=[
            pltpu.VMEM((_CHUNK,), jnp.int32),
            pltpu.VMEM((_CHUNK,), jnp.int32),
            pltpu.VMEM((_CHUNK, _H), jnp.float32),
            pltpu.VMEM_SHARED((_NPAD, _H), jnp.float32),
            pltpu.SemaphoreType.DMA,
        ],
    )
    def edge_acc_kernel(hs_hbm, src_hbm, dst_hbm, zeros_hbm, out_hbm,
                        src_v, dst_v, rows_v, acc_sh, sem):
        c = lax.axis_index("c")
        s = lax.axis_index("s")
        w = s * _NC + c
        sl = pl.ds(s * _RPS, _RPS)
        pltpu.sync_copy(zeros_hbm.at[sl], acc_sh.at[sl])
        plsc.subcore_barrier()

        @pl.loop(0, _chunks_for(w))
        def _(t):
            base = (w + t * _NW) * _CHUNK
            pltpu.sync_copy(src_hbm.at[pl.ds(base, _CHUNK)], src_v)
            pltpu.sync_copy(dst_hbm.at[pl.ds(base, _CHUNK)], dst_v)
            pltpu.async_copy(hs_hbm.at[src_v], rows_v, sem).wait()
            pltpu.sync_copy(rows_v, acc_sh.at[dst_v], add=True)

        plsc.subcore_barrier()
        pltpu.sync_copy(acc_sh.at[sl], out_hbm.at[c, sl])

    return deg_kernel, edge_acc_kernel


def _phase2_body(x_ref, dacc_ref, w1_ref, hs_ref, dinv_ref):
    deg = dacc_ref[0, :_N, 0:1] + dacc_ref[1, :_N, 0:1] + 1.0
    dinv = lax.rsqrt(deg)
    h = jnp.dot(x_ref[...], w1_ref[...], preferred_element_type=jnp.float32)
    hs_ref[...] = h * dinv
    dinv_ref[...] = dinv


def _phase4_body(acc_ref, hs1_ref, dinv_ref, b1_ref, w2_ref, hs2_ref):
    dinv = dinv_ref[...]
    h = (acc_ref[0, :_N] + acc_ref[1, :_N] + hs1_ref[...]) * dinv + b1_ref[...]
    h = jnp.maximum(h, 0.0)
    hs2_ref[...] = jnp.dot(h, w2_ref[...], preferred_element_type=jnp.float32) * dinv


def _phase6_body(acc_ref, hs2_ref, dinv_ref, b2_ref, batch_ref, maskf_ref,
                 wout_ref, bout_ref, out_ref):
    dinv = dinv_ref[...]
    h = (acc_ref[0, :_N] + acc_ref[1, :_N] + hs2_ref[...]) * dinv + b2_ref[...]
    h = jnp.maximum(h, 0.0)
    gids = lax.broadcasted_iota(jnp.int32, (_G, _N), 0)
    in_g = (batch_ref[...] == gids).astype(jnp.float32)
    in_g_m = in_g * maskf_ref[...]
    ent_sum = jnp.dot(in_g_m, h, preferred_element_type=jnp.float32)
    pool_sum = jnp.dot(in_g, h, preferred_element_type=jnp.float32)
    ent_cnt = jnp.sum(in_g_m, axis=1, keepdims=True)
    node_cnt = jnp.sum(in_g, axis=1, keepdims=True)
    doc = jnp.where(ent_cnt < _EPS,
                    pool_sum / jnp.maximum(node_cnt, 1.0),
                    ent_sum / (ent_cnt + _EPS))
    logits = jnp.dot(doc, wout_ref[...], preferred_element_type=jnp.float32)
    logits = logits + bout_ref[...]
    zmax = jnp.max(logits, axis=1, keepdims=True)
    z = logits - zmax
    out_ref[...] = z - jnp.log(jnp.sum(jnp.exp(z), axis=1, keepdims=True))


def kernel(x_fp16, ei, batch, mask, W1, b1, W2, b2, Wout, bout):
    x_f32 = x_fp16.astype(jnp.float32)
    src = ei[0].astype(jnp.int32)
    dst = ei[1].astype(jnp.int32)
    batch_i = batch.astype(jnp.int32).reshape(1, _N)
    maskf = mask.astype(jnp.float32).reshape(1, _N)
    onesH = jnp.ones((_CHUNK, _H), jnp.float32)
    zerosH = jnp.zeros((_NPAD, _H), jnp.float32)

    deg_kernel, edge_acc_kernel = _sc_kernels()
    dacc = deg_kernel(dst, onesH, zerosH)

    hs1, dinv = pl.pallas_call(
        _phase2_body,
        out_shape=(
            jax.ShapeDtypeStruct((_N, _H), jnp.float32),
            jax.ShapeDtypeStruct((_N, 1), jnp.float32),
        ),
    )(x_f32, dacc, W1)

    acc1 = edge_acc_kernel(hs1, src, dst, zerosH)

    hs2 = pl.pallas_call(
        _phase4_body,
        out_shape=jax.ShapeDtypeStruct((_N, _H), jnp.float32),
    )(acc1, hs1, dinv, b1.reshape(1, _H), W2)

    acc2 = edge_acc_kernel(hs2, src, dst, zerosH)

    out = pl.pallas_call(
        _phase6_body,
        out_shape=jax.ShapeDtypeStruct((_G, bout.shape[0]), jnp.float32),
    )(acc2, hs2, dinv, b2.reshape(1, _H), batch_i, maskf, Wout,
      bout.reshape(1, bout.shape[0]))
    return out


# speedup vs baseline: 16.9350x; 1.3355x over previous
import functools

import jax
import jax.numpy as jnp
from jax import lax
from jax.experimental import pallas as pl
from jax.experimental.pallas import tpu as pltpu
from jax.experimental.pallas import tpu_sc as plsc

_N = 10000
_E = 160000
_H = 128
_G = 64
_EPS = 1e-6

_NC, _NS = 2, 16
_NW = _NC * _NS
_CHUNK = 128
_NCHUNKS = _E // _CHUNK
_NPAD = 10112
_RPS = _NPAD // _NS

_S = 3
_NG = 13
_TAIL = _NCHUNKS - _NG * _S * _NW


@functools.cache
def _sc_kernels():
    mesh = plsc.VectorSubcoreMesh(
        core_axis_name="c", subcore_axis_name="s",
        num_cores=_NC, num_subcores=_NS,
    )

    @functools.partial(
        pl.kernel,
        out_type=jax.ShapeDtypeStruct((_NC, _NPAD, _H), jnp.float32),
        mesh=mesh,
        scratch_types=[
            [pltpu.VMEM((_CHUNK,), jnp.int32) for _ in range(_S)],
            pltpu.VMEM((_CHUNK, _H), jnp.float32),
            pltpu.VMEM_SHARED((_NPAD, _H), jnp.float32),
            pltpu.SemaphoreType.DMA,
            pltpu.SemaphoreType.DMA,
        ],
    )
    def deg_kernel(dst_hbm, ones_hbm, zeros_hbm, out_hbm,
                   dst_v, ones_v, acc_sh, isem, ssem):
        c = lax.axis_index("c")
        s = lax.axis_index("s")
        w = s * _NC + c
        sl = pl.ds(s * _RPS, _RPS)
        pltpu.sync_copy(zeros_hbm.at[sl], acc_sh.at[sl])
        pltpu.sync_copy(ones_hbm, ones_v)
        plsc.subcore_barrier()

        @pl.loop(0, _NG)
        def _(g):
            loads = []
            for j in range(_S):
                base = (w + (g * _S + j) * _NW) * _CHUNK
                loads.append(pltpu.async_copy(
                    dst_hbm.at[pl.ds(base, _CHUNK)], dst_v[j], isem))
            for d in loads:
                d.wait()
            stores = [pltpu.async_copy(ones_v, acc_sh.at[dst_v[j]], ssem,
                                       add=True) for j in range(_S)]
            for d in stores:
                d.wait()

        @pl.when(w < _TAIL)
        def _():
            base = (_NG * _S * _NW + w) * _CHUNK
            pltpu.sync_copy(dst_hbm.at[pl.ds(base, _CHUNK)], dst_v[0])
            pltpu.sync_copy(ones_v, acc_sh.at[dst_v[0]], add=True)

        plsc.subcore_barrier()
        pltpu.sync_copy(acc_sh.at[sl], out_hbm.at[c, sl])

    @functools.partial(
        pl.kernel,
        out_type=jax.ShapeDtypeStruct((_NC, _NPAD, _H), jnp.float32),
        mesh=mesh,
        scratch_types=[
            [pltpu.VMEM((_CHUNK,), jnp.int32) for _ in range(_S)],
            [pltpu.VMEM((_CHUNK,), jnp.int32) for _ in range(_S)],
            [pltpu.VMEM((_CHUNK, _H), jnp.float32) for _ in range(_S)],
            pltpu.VMEM_SHARED((_NPAD, _H), jnp.float32),
            pltpu.SemaphoreType.DMA,
            pltpu.SemaphoreType.DMA,
            pltpu.SemaphoreType.DMA,
        ],
    )
    def edge_acc_kernel(hs_hbm, src_hbm, dst_hbm, zeros_hbm, out_hbm,
                        src_v, dst_v, rows_v, acc_sh, isem, gsem, ssem):
        c = lax.axis_index("c")
        s = lax.axis_index("s")
        w = s * _NC + c
        sl = pl.ds(s * _RPS, _RPS)
        pltpu.sync_copy(zeros_hbm.at[sl], acc_sh.at[sl])
        plsc.subcore_barrier()

        @pl.loop(0, _NG)
        def _(g):
            loads = []
            for j in range(_S):
                base = (w + (g * _S + j) * _NW) * _CHUNK
                loads.append(pltpu.async_copy(
                    src_hbm.at[pl.ds(base, _CHUNK)], src_v[j], isem))
                loads.append(pltpu.async_copy(
                    dst_hbm.at[pl.ds(base, _CHUNK)], dst_v[j], isem))
            for d in loads:
                d.wait()
            gathers = [pltpu.async_copy(hs_hbm.at[src_v[j]], rows_v[j], gsem)
                       for j in range(_S)]
            for d in gathers:
                d.wait()
            stores = [pltpu.async_copy(rows_v[j], acc_sh.at[dst_v[j]], ssem,
                                       add=True) for j in range(_S)]
            for d in stores:
                d.wait()

        @pl.when(w < _TAIL)
        def _():
            base = (_NG * _S * _NW + w) * _CHUNK
            pltpu.sync_copy(src_hbm.at[pl.ds(base, _CHUNK)], src_v[0])
            pltpu.sync_copy(dst_hbm.at[pl.ds(base, _CHUNK)], dst_v[0])
            pltpu.async_copy(hs_hbm.at[src_v[0]], rows_v[0], gsem).wait()
            pltpu.sync_copy(rows_v[0], acc_sh.at[dst_v[0]], add=True)

        plsc.subcore_barrier()
        pltpu.sync_copy(acc_sh.at[sl], out_hbm.at[c, sl])

    return deg_kernel, edge_acc_kernel


def _phase2_body(x_ref, dacc_ref, w1_ref, hs_ref, dinv_ref):
    deg = dacc_ref[0, :_N, 0:1] + dacc_ref[1, :_N, 0:1] + 1.0
    dinv = lax.rsqrt(deg)
    h = jnp.dot(x_ref[...], w1_ref[...], preferred_element_type=jnp.float32)
    hs_ref[...] = h * dinv
    dinv_ref[...] = dinv


def _phase4_body(acc_ref, hs1_ref, dinv_ref, b1_ref, w2_ref, hs2_ref):
    dinv = dinv_ref[...]
    h = (acc_ref[0, :_N] + acc_ref[1, :_N] + hs1_ref[...]) * dinv + b1_ref[...]
    h = jnp.maximum(h, 0.0)
    hs2_ref[...] = jnp.dot(h, w2_ref[...], preferred_element_type=jnp.float32) * dinv


def _phase6_body(acc_ref, hs2_ref, dinv_ref, b2_ref, batch_ref, maskf_ref,
                 wout_ref, bout_ref, out_ref):
    dinv = dinv_ref[...]
    h = (acc_ref[0, :_N] + acc_ref[1, :_N] + hs2_ref[...]) * dinv + b2_ref[...]
    h = jnp.maximum(h, 0.0)
    gids = lax.broadcasted_iota(jnp.int32, (_G, _N), 0)
    in_g = (batch_ref[...] == gids).astype(jnp.float32)
    in_g_m = in_g * maskf_ref[...]
    ent_sum = jnp.dot(in_g_m, h, preferred_element_type=jnp.float32)
    pool_sum = jnp.dot(in_g, h, preferred_element_type=jnp.float32)
    ent_cnt = jnp.sum(in_g_m, axis=1, keepdims=True)
    node_cnt = jnp.sum(in_g, axis=1, keepdims=True)
    doc = jnp.where(ent_cnt < _EPS,
                    pool_sum / jnp.maximum(node_cnt, 1.0),
                    ent_sum / (ent_cnt + _EPS))
    logits = jnp.dot(doc, wout_ref[...], preferred_element_type=jnp.float32)
    logits = logits + bout_ref[...]
    zmax = jnp.max(logits, axis=1, keepdims=True)
    z = logits - zmax
    out_ref[...] = z - jnp.log(jnp.sum(jnp.exp(z), axis=1, keepdims=True))


def kernel(x_fp16, ei, batch, mask, W1, b1, W2, b2, Wout, bout):
    x_f32 = x_fp16.astype(jnp.float32)
    src = ei[0].astype(jnp.int32)
    dst = ei[1].astype(jnp.int32)
    batch_i = batch.astype(jnp.int32).reshape(1, _N)
    maskf = mask.astype(jnp.float32).reshape(1, _N)
    onesH = jnp.ones((_CHUNK, _H), jnp.float32)
    zerosH = jnp.zeros((_NPAD, _H), jnp.float32)

    deg_kernel, edge_acc_kernel = _sc_kernels()
    dacc = deg_kernel(dst, onesH, zerosH)

    hs1, dinv = pl.pallas_call(
        _phase2_body,
        out_shape=(
            jax.ShapeDtypeStruct((_N, _H), jnp.float32),
            jax.ShapeDtypeStruct((_N, 1), jnp.float32),
        ),
    )(x_f32, dacc, W1)

    acc1 = edge_acc_kernel(hs1, src, dst, zerosH)

    hs2 = pl.pallas_call(
        _phase4_body,
        out_shape=jax.ShapeDtypeStruct((_N, _H), jnp.float32),
    )(acc1, hs1, dinv, b1.reshape(1, _H), W2)

    acc2 = edge_acc_kernel(hs2, src, dst, zerosH)

    out = pl.pallas_call(
        _phase6_body,
        out_shape=jax.ShapeDtypeStruct((_G, bout.shape[0]), jnp.float32),
    )(acc2, hs2, dinv, b2.reshape(1, _H), batch_i, maskf, Wout,
      bout.reshape(1, bout.shape[0]))
    return out


# speedup vs baseline: 18.5224x; 1.0937x over previous
import functools

import jax
import jax.numpy as jnp
from jax import lax
from jax.experimental import pallas as pl
from jax.experimental.pallas import tpu as pltpu
from jax.experimental.pallas import tpu_sc as plsc

_N = 10000
_E = 160000
_H = 128
_G = 64
_EPS = 1e-6

_NC, _NS = 2, 16
_NW = _NC * _NS
_CHUNK = 128
_NCHUNKS = _E // _CHUNK
_NPAD = 10112
_RPS = _NPAD // _NS

_S = 3
_NG = 13
_TAIL = _NCHUNKS - _NG * _S * _NW


@functools.cache
def _sc_kernels():
    mesh = plsc.VectorSubcoreMesh(
        core_axis_name="c", subcore_axis_name="s",
        num_cores=_NC, num_subcores=_NS,
    )

    @functools.partial(
        pl.kernel,
        out_type=jax.ShapeDtypeStruct((_NC, _NPAD, _H), jnp.float32),
        mesh=mesh,
        scratch_types=[
            [pltpu.VMEM((_CHUNK,), jnp.int32) for _ in range(_S)],
            pltpu.VMEM((_CHUNK, _H), jnp.float32),
            pltpu.VMEM_SHARED((_NPAD, _H), jnp.float32),
            [pltpu.SemaphoreType.DMA for _ in range(_S)],
            [pltpu.SemaphoreType.DMA for _ in range(_S)],
        ],
    )
    def deg_kernel(dst_hbm, ones_hbm, zeros_hbm, out_hbm,
                   dst_v, ones_v, acc_sh, isem, ssem):
        c = lax.axis_index("c")
        s = lax.axis_index("s")
        w = s * _NC + c
        sl = pl.ds(s * _RPS, _RPS)
        pltpu.sync_copy(zeros_hbm.at[sl], acc_sh.at[sl])
        pltpu.sync_copy(ones_hbm, ones_v)
        plsc.subcore_barrier()

        def drain_scatter(j):
            pltpu.make_async_copy(ones_v, acc_sh.at[dst_v[j]], ssem[j]).wait()

        @pl.loop(0, _NG)
        def _(g):
            idx = []
            for j in range(_S):
                @pl.when(g > 0)
                def _():
                    drain_scatter(j)
                base = (w + (g * _S + j) * _NW) * _CHUNK
                idx.append(pltpu.async_copy(
                    dst_hbm.at[pl.ds(base, _CHUNK)], dst_v[j], isem[j]))
            for j in range(_S):
                idx[j].wait()
                pltpu.async_copy(ones_v, acc_sh.at[dst_v[j]], ssem[j],
                                 add=True)

        for j in range(_S):
            drain_scatter(j)

        @pl.when(w < _TAIL)
        def _():
            base = (_NG * _S * _NW + w) * _CHUNK
            pltpu.sync_copy(dst_hbm.at[pl.ds(base, _CHUNK)], dst_v[0])
            pltpu.sync_copy(ones_v, acc_sh.at[dst_v[0]], add=True)

        plsc.subcore_barrier()
        pltpu.sync_copy(acc_sh.at[sl], out_hbm.at[c, sl])

    @functools.partial(
        pl.kernel,
        out_type=jax.ShapeDtypeStruct((_NC, _NPAD, _H), jnp.float32),
        mesh=mesh,
        scratch_types=[
            [pltpu.VMEM((_CHUNK,), jnp.int32) for _ in range(_S)],
            [pltpu.VMEM((_CHUNK,), jnp.int32) for _ in range(_S)],
            [pltpu.VMEM((_CHUNK, _H), jnp.float32) for _ in range(_S)],
            pltpu.VMEM_SHARED((_NPAD, _H), jnp.float32),
            [pltpu.SemaphoreType.DMA for _ in range(_S)],
            [pltpu.SemaphoreType.DMA for _ in range(_S)],
            [pltpu.SemaphoreType.DMA for _ in range(_S)],
        ],
    )
    def edge_acc_kernel(hs_hbm, src_hbm, dst_hbm, zeros_hbm, out_hbm,
                        src_v, dst_v, rows_v, acc_sh, isem, gsem, ssem):
        c = lax.axis_index("c")
        s = lax.axis_index("s")
        w = s * _NC + c
        sl = pl.ds(s * _RPS, _RPS)
        pltpu.sync_copy(zeros_hbm.at[sl], acc_sh.at[sl])
        plsc.subcore_barrier()

        def drain_scatter(j):
            pltpu.make_async_copy(rows_v[j], acc_sh.at[dst_v[j]],
                                  ssem[j]).wait()

        @pl.loop(0, _NG)
        def _(g):
            idx = []
            for j in range(_S):
                @pl.when(g > 0)
                def _():
                    drain_scatter(j)
                base = (w + (g * _S + j) * _NW) * _CHUNK
                idx.append((
                    pltpu.async_copy(src_hbm.at[pl.ds(base, _CHUNK)],
                                     src_v[j], isem[j]),
                    pltpu.async_copy(dst_hbm.at[pl.ds(base, _CHUNK)],
                                     dst_v[j], isem[j]),
                ))
            gathers = []
            for j in range(_S):
                idx[j][0].wait()
                idx[j][1].wait()
                gathers.append(pltpu.async_copy(hs_hbm.at[src_v[j]],
                                                rows_v[j], gsem[j]))
            for j in range(_S):
                gathers[j].wait()
                pltpu.async_copy(rows_v[j], acc_sh.at[dst_v[j]], ssem[j],
                                 add=True)

        for j in range(_S):
            drain_scatter(j)

        @pl.when(w < _TAIL)
        def _():
            base = (_NG * _S * _NW + w) * _CHUNK
            pltpu.sync_copy(src_hbm.at[pl.ds(base, _CHUNK)], src_v[0])
            pltpu.sync_copy(dst_hbm.at[pl.ds(base, _CHUNK)], dst_v[0])
            pltpu.async_copy(hs_hbm.at[src_v[0]], rows_v[0], gsem[0]).wait()
            pltpu.sync_copy(rows_v[0], acc_sh.at[dst_v[0]], add=True)

        plsc.subcore_barrier()
        pltpu.sync_copy(acc_sh.at[sl], out_hbm.at[c, sl])

    return deg_kernel, edge_acc_kernel


def _phase2_body(x_ref, dacc_ref, w1_ref, hs_ref, dinv_ref):
    deg = dacc_ref[0, :_N, 0:1] + dacc_ref[1, :_N, 0:1] + 1.0
    dinv = lax.rsqrt(deg)
    h = jnp.dot(x_ref[...], w1_ref[...], preferred_element_type=jnp.float32)
    hs_ref[...] = h * dinv
    dinv_ref[...] = dinv


def _phase4_body(acc_ref, hs1_ref, dinv_ref, b1_ref, w2_ref, hs2_ref):
    dinv = dinv_ref[...]
    h = (acc_ref[0, :_N] + acc_ref[1, :_N] + hs1_ref[...]) * dinv + b1_ref[...]
    h = jnp.maximum(h, 0.0)
    hs2_ref[...] = jnp.dot(h, w2_ref[...], preferred_element_type=jnp.float32) * dinv


def _phase6_body(acc_ref, hs2_ref, dinv_ref, b2_ref, batch_ref, maskf_ref,
                 wout_ref, bout_ref, out_ref):
    dinv = dinv_ref[...]
    h = (acc_ref[0, :_N] + acc_ref[1, :_N] + hs2_ref[...]) * dinv + b2_ref[...]
    h = jnp.maximum(h, 0.0)
    gids = lax.broadcasted_iota(jnp.int32, (_G, _N), 0)
    in_g = (batch_ref[...] == gids).astype(jnp.float32)
    in_g_m = in_g * maskf_ref[...]
    ent_sum = jnp.dot(in_g_m, h, preferred_element_type=jnp.float32)
    pool_sum = jnp.dot(in_g, h, preferred_element_type=jnp.float32)
    ent_cnt = jnp.sum(in_g_m, axis=1, keepdims=True)
    node_cnt = jnp.sum(in_g, axis=1, keepdims=True)
    doc = jnp.where(ent_cnt < _EPS,
                    pool_sum / jnp.maximum(node_cnt, 1.0),
                    ent_sum / (ent_cnt + _EPS))
    logits = jnp.dot(doc, wout_ref[...], preferred_element_type=jnp.float32)
    logits = logits + bout_ref[...]
    zmax = jnp.max(logits, axis=1, keepdims=True)
    z = logits - zmax
    out_ref[...] = z - jnp.log(jnp.sum(jnp.exp(z), axis=1, keepdims=True))


def kernel(x_fp16, ei, batch, mask, W1, b1, W2, b2, Wout, bout):
    x_f32 = x_fp16.astype(jnp.float32)
    src = ei[0].astype(jnp.int32)
    dst = ei[1].astype(jnp.int32)
    batch_i = batch.astype(jnp.int32).reshape(1, _N)
    maskf = mask.astype(jnp.float32).reshape(1, _N)
    onesH = jnp.ones((_CHUNK, _H), jnp.float32)
    zerosH = jnp.zeros((_NPAD, _H), jnp.float32)

    deg_kernel, edge_acc_kernel = _sc_kernels()
    dacc = deg_kernel(dst, onesH, zerosH)

    hs1, dinv = pl.pallas_call(
        _phase2_body,
        out_shape=(
            jax.ShapeDtypeStruct((_N, _H), jnp.float32),
            jax.ShapeDtypeStruct((_N, 1), jnp.float32),
        ),
    )(x_f32, dacc, W1)

    acc1 = edge_acc_kernel(hs1, src, dst, zerosH)

    hs2 = pl.pallas_call(
        _phase4_body,
        out_shape=jax.ShapeDtypeStruct((_N, _H), jnp.float32),
    )(acc1, hs1, dinv, b1.reshape(1, _H), W2)

    acc2 = edge_acc_kernel(hs2, src, dst, zerosH)

    out = pl.pallas_call(
        _phase6_body,
        out_shape=jax.ShapeDtypeStruct((_G, bout.shape[0]), jnp.float32),
    )(acc2, hs2, dinv, b2.reshape(1, _H), batch_i, maskf, Wout,
      bout.reshape(1, bout.shape[0]))
    return out


# speedup vs baseline: 18.6964x; 1.0094x over previous
import functools

import jax
import jax.numpy as jnp
from jax import lax
from jax.experimental import pallas as pl
from jax.experimental.pallas import tpu as pltpu
from jax.experimental.pallas import tpu_sc as plsc

_N = 10000
_E = 160000
_H = 128
_G = 64
_EPS = 1e-6

_NC, _NS = 2, 16
_NW = _NC * _NS
_CHUNK = 128
_NCHUNKS = _E // _CHUNK
_NPAD = 10112
_RPS = _NPAD // _NS

_S = 3
_NG = 13
_TAIL = _NCHUNKS - _NG * _S * _NW


@functools.cache
def _sc_kernels():
    mesh = plsc.VectorSubcoreMesh(
        core_axis_name="c", subcore_axis_name="s",
        num_cores=_NC, num_subcores=_NS,
    )

    @functools.partial(
        pl.kernel,
        out_type=jax.ShapeDtypeStruct((_NC, _NPAD, _H), jnp.float32),
        mesh=mesh,
        scratch_types=[
            [pltpu.VMEM((_CHUNK,), jnp.int32) for _ in range(_S)],
            pltpu.VMEM((_CHUNK, _H), jnp.float32),
            pltpu.VMEM_SHARED((_NPAD, _H), jnp.float32),
            [pltpu.SemaphoreType.DMA for _ in range(_S)],
            [pltpu.SemaphoreType.DMA for _ in range(_S)],
        ],
    )
    def deg_kernel(dst_hbm, ones_hbm, zeros_hbm, out_hbm,
                   dst_v, ones_v, acc_sh, isem, ssem):
        c = lax.axis_index("c")
        s = lax.axis_index("s")
        w = s * _NC + c
        sl = pl.ds(s * _RPS, _RPS)
        pltpu.sync_copy(zeros_hbm.at[sl], acc_sh.at[sl])
        pltpu.sync_copy(ones_hbm, ones_v)
        plsc.subcore_barrier()

        def drain_scatter(j):
            pltpu.make_async_copy(ones_v, acc_sh.at[dst_v[j]], ssem[j]).wait()

        @pl.loop(0, _NG)
        def _(g):
            idx = []
            for j in range(_S):
                @pl.when(g > 0)
                def _():
                    drain_scatter(j)
                base = (w + (g * _S + j) * _NW) * _CHUNK
                idx.append(pltpu.async_copy(
                    dst_hbm.at[pl.ds(base, _CHUNK)], dst_v[j], isem[j]))
            for j in range(_S):
                idx[j].wait()
                pltpu.async_copy(ones_v, acc_sh.at[dst_v[j]], ssem[j],
                                 add=True)

        for j in range(_S):
            drain_scatter(j)

        @pl.when(w < _TAIL)
        def _():
            base = (_NG * _S * _NW + w) * _CHUNK
            pltpu.sync_copy(dst_hbm.at[pl.ds(base, _CHUNK)], dst_v[0])
            pltpu.sync_copy(ones_v, acc_sh.at[dst_v[0]], add=True)

        plsc.subcore_barrier()
        pltpu.sync_copy(acc_sh.at[sl], out_hbm.at[c, sl])

    @functools.partial(
        pl.kernel,
        out_type=jax.ShapeDtypeStruct((_NC, _NPAD, _H), jnp.float32),
        mesh=mesh,
        scratch_types=[
            [pltpu.VMEM((_CHUNK,), jnp.int32) for _ in range(_S)],
            [pltpu.VMEM((_CHUNK,), jnp.int32) for _ in range(_S)],
            [pltpu.VMEM((_CHUNK, _H), jnp.float32) for _ in range(_S)],
            pltpu.VMEM_SHARED((_NPAD, _H), jnp.float32),
            [pltpu.SemaphoreType.DMA for _ in range(_S)],
            [pltpu.SemaphoreType.DMA for _ in range(_S)],
            [pltpu.SemaphoreType.DMA for _ in range(_S)],
        ],
    )
    def edge_acc_kernel(hs_hbm, src_hbm, dst_hbm, zeros_hbm, out_hbm,
                        src_v, dst_v, rows_v, acc_sh, isem, gsem, ssem):
        c = lax.axis_index("c")
        s = lax.axis_index("s")
        w = s * _NC + c
        sl = pl.ds(s * _RPS, _RPS)
        pltpu.sync_copy(zeros_hbm.at[sl], acc_sh.at[sl])
        plsc.subcore_barrier()

        def drain_scatter(j):
            pltpu.make_async_copy(rows_v[j], acc_sh.at[dst_v[j]],
                                  ssem[j]).wait()

        @pl.loop(0, _NG)
        def _(g):
            idx = []
            for j in range(_S):
                @pl.when(g > 0)
                def _():
                    drain_scatter(j)
                base = (w + (g * _S + j) * _NW) * _CHUNK
                idx.append((
                    pltpu.async_copy(src_hbm.at[pl.ds(base, _CHUNK)],
                                     src_v[j], isem[j]),
                    pltpu.async_copy(dst_hbm.at[pl.ds(base, _CHUNK)],
                                     dst_v[j], isem[j]),
                ))
            gathers = []
            for j in range(_S):
                idx[j][0].wait()
                idx[j][1].wait()
                gathers.append(pltpu.async_copy(hs_hbm.at[src_v[j]],
                                                rows_v[j], gsem[j]))
            for j in range(_S):
                gathers[j].wait()
                pltpu.async_copy(rows_v[j], acc_sh.at[dst_v[j]], ssem[j],
                                 add=True)

        for j in range(_S):
            drain_scatter(j)

        @pl.when(w < _TAIL)
        def _():
            base = (_NG * _S * _NW + w) * _CHUNK
            pltpu.sync_copy(src_hbm.at[pl.ds(base, _CHUNK)], src_v[0])
            pltpu.sync_copy(dst_hbm.at[pl.ds(base, _CHUNK)], dst_v[0])
            pltpu.async_copy(hs_hbm.at[src_v[0]], rows_v[0], gsem[0]).wait()
            pltpu.sync_copy(rows_v[0], acc_sh.at[dst_v[0]], add=True)

        plsc.subcore_barrier()
        pltpu.sync_copy(acc_sh.at[sl], out_hbm.at[c, sl])

    return deg_kernel, edge_acc_kernel


def _matmul_body(x_ref, w1_ref, h_ref):
    h_ref[...] = jnp.dot(x_ref[...], w1_ref[...],
                         preferred_element_type=jnp.float32)


def _phase2_body(h_ref, dacc_ref, hs_ref, dinv_ref):
    deg = dacc_ref[0, :_N, 0:1] + dacc_ref[1, :_N, 0:1] + 1.0
    dinv = lax.rsqrt(deg)
    hs_ref[...] = h_ref[...] * dinv
    dinv_ref[...] = dinv


def _phase4_body(acc_ref, hs1_ref, dinv_ref, b1_ref, w2_ref, hs2_ref):
    dinv = dinv_ref[...]
    h = (acc_ref[0, :_N] + acc_ref[1, :_N] + hs1_ref[...]) * dinv + b1_ref[...]
    h = jnp.maximum(h, 0.0)
    hs2_ref[...] = jnp.dot(h, w2_ref[...], preferred_element_type=jnp.float32) * dinv


def _phase6_body(acc_ref, hs2_ref, dinv_ref, b2_ref, batch_ref, maskf_ref,
                 wout_ref, bout_ref, out_ref):
    dinv = dinv_ref[...]
    h = (acc_ref[0, :_N] + acc_ref[1, :_N] + hs2_ref[...]) * dinv + b2_ref[...]
    h = jnp.maximum(h, 0.0)
    gids = lax.broadcasted_iota(jnp.int32, (_G, _N), 0)
    in_g = (batch_ref[...] == gids).astype(jnp.float32)
    in_g_m = in_g * maskf_ref[...]
    ent_sum = jnp.dot(in_g_m, h, preferred_element_type=jnp.float32)
    pool_sum = jnp.dot(in_g, h, preferred_element_type=jnp.float32)
    ent_cnt = jnp.sum(in_g_m, axis=1, keepdims=True)
    node_cnt = jnp.sum(in_g, axis=1, keepdims=True)
    doc = jnp.where(ent_cnt < _EPS,
                    pool_sum / jnp.maximum(node_cnt, 1.0),
                    ent_sum / (ent_cnt + _EPS))
    logits = jnp.dot(doc, wout_ref[...], preferred_element_type=jnp.float32)
    logits = logits + bout_ref[...]
    zmax = jnp.max(logits, axis=1, keepdims=True)
    z = logits - zmax
    out_ref[...] = z - jnp.log(jnp.sum(jnp.exp(z), axis=1, keepdims=True))


def kernel(x_fp16, ei, batch, mask, W1, b1, W2, b2, Wout, bout):
    x_f32 = x_fp16.astype(jnp.float32)
    src = ei[0].astype(jnp.int32)
    dst = ei[1].astype(jnp.int32)
    batch_i = batch.astype(jnp.int32).reshape(1, _N)
    maskf = mask.astype(jnp.float32).reshape(1, _N)
    onesH = jnp.ones((_CHUNK, _H), jnp.float32)
    zerosH = jnp.zeros((_NPAD, _H), jnp.float32)

    deg_kernel, edge_acc_kernel = _sc_kernels()
    dacc = deg_kernel(dst, onesH, zerosH)

    h1 = pl.pallas_call(
        _matmul_body,
        out_shape=jax.ShapeDtypeStruct((_N, _H), jnp.float32),
    )(x_f32, W1)

    hs1, dinv = pl.pallas_call(
        _phase2_body,
        out_shape=(
            jax.ShapeDtypeStruct((_N, _H), jnp.float32),
            jax.ShapeDtypeStruct((_N, 1), jnp.float32),
        ),
    )(h1, dacc)

    acc1 = edge_acc_kernel(hs1, src, dst, zerosH)

    hs2 = pl.pallas_call(
        _phase4_body,
        out_shape=jax.ShapeDtypeStruct((_N, _H), jnp.float32),
    )(acc1, hs1, dinv, b1.reshape(1, _H), W2)

    acc2 = edge_acc_kernel(hs2, src, dst, zerosH)

    out = pl.pallas_call(
        _phase6_body,
        out_shape=jax.ShapeDtypeStruct((_G, bout.shape[0]), jnp.float32),
    )(acc2, hs2, dinv, b2.reshape(1, _H), batch_i, maskf, Wout,
      bout.reshape(1, bout.shape[0]))
    return out


# speedup vs baseline: 19.6687x; 1.0520x over previous
import functools

import jax
import jax.numpy as jnp
from jax import lax
from jax.experimental import pallas as pl
from jax.experimental.pallas import tpu as pltpu
from jax.experimental.pallas import tpu_sc as plsc

_N = 10000
_E = 160000
_H = 128
_G = 64
_EPS = 1e-6

_NC, _NS = 2, 16
_NW = _NC * _NS
_CHUNK = 128
_NCHUNKS = _E // _CHUNK
_RPS = 632
_RPS_LAST = _N - (_NS - 1) * _RPS

_S = 3
_NG = 13
_TAIL = _NCHUNKS - _NG * _S * _NW


@functools.cache
def _sc_kernels():
    mesh = plsc.VectorSubcoreMesh(
        core_axis_name="c", subcore_axis_name="s",
        num_cores=_NC, num_subcores=_NS,
    )

    @functools.partial(
        pl.kernel,
        out_type=jax.ShapeDtypeStruct((_NC, _N, _H), jnp.float32),
        mesh=mesh,
        scratch_types=[
            [pltpu.VMEM((_CHUNK,), jnp.int32) for _ in range(_S)],
            pltpu.VMEM((_CHUNK, _H), jnp.float32),
            pltpu.VMEM_SHARED((_N, _H), jnp.float32),
            [pltpu.SemaphoreType.DMA for _ in range(_S)],
            [pltpu.SemaphoreType.DMA for _ in range(_S)],
        ],
    )
    def deg_kernel(dst_hbm, ones_hbm, zeros_hbm, out_hbm,
                   dst_v, ones_v, acc_sh, isem, ssem):
        c = lax.axis_index("c")
        s = lax.axis_index("s")
        w = s * _NC + c

        def my_rows(fn):
            @pl.when(s < _NS - 1)
            def _():
                fn(pl.ds(s * _RPS, _RPS))

            @pl.when(s == _NS - 1)
            def _():
                fn(pl.ds((_NS - 1) * _RPS, _RPS_LAST))

        my_rows(lambda sl: pltpu.sync_copy(zeros_hbm.at[sl], acc_sh.at[sl]))
        pltpu.sync_copy(ones_hbm, ones_v)
        plsc.subcore_barrier()

        def drain_scatter(j):
            pltpu.make_async_copy(ones_v, acc_sh.at[dst_v[j]], ssem[j]).wait()

        @pl.loop(0, _NG)
        def _(g):
            idx = []
            for j in range(_S):
                @pl.when(g > 0)
                def _():
                    drain_scatter(j)
                base = (w + (g * _S + j) * _NW) * _CHUNK
                idx.append(pltpu.async_copy(
                    dst_hbm.at[pl.ds(base, _CHUNK)], dst_v[j], isem[j]))
            for j in range(_S):
                idx[j].wait()
                pltpu.async_copy(ones_v, acc_sh.at[dst_v[j]], ssem[j],
                                 add=True)

        for j in range(_S):
            drain_scatter(j)

        @pl.when(w < _TAIL)
        def _():
            base = (_NG * _S * _NW + w) * _CHUNK
            pltpu.sync_copy(dst_hbm.at[pl.ds(base, _CHUNK)], dst_v[0])
            pltpu.sync_copy(ones_v, acc_sh.at[dst_v[0]], add=True)

        plsc.subcore_barrier()
        my_rows(lambda sl: pltpu.sync_copy(acc_sh.at[sl], out_hbm.at[c, sl]))

    @functools.partial(
        pl.kernel,
        out_type=jax.ShapeDtypeStruct((_NC, _N, _H), jnp.float32),
        mesh=mesh,
        scratch_types=[
            [pltpu.VMEM((_CHUNK,), jnp.int32) for _ in range(2 * _S)],
            [pltpu.VMEM((_CHUNK,), jnp.int32) for _ in range(2 * _S)],
            [pltpu.VMEM((_CHUNK, _H), jnp.float32) for _ in range(_S)],
            pltpu.VMEM_SHARED((_N, _H), jnp.float32),
            [pltpu.SemaphoreType.DMA for _ in range(2 * _S)],
            [pltpu.SemaphoreType.DMA for _ in range(_S)],
            [pltpu.SemaphoreType.DMA for _ in range(_S)],
        ],
    )
    def edge_acc_kernel(hs_hbm, src_hbm, dst_hbm, zeros_hbm, out_hbm,
                        src_v, dst_v, rows_v, acc_sh, isem, gsem, ssem):
        c = lax.axis_index("c")
        s = lax.axis_index("s")
        w = s * _NC + c

        def my_rows(fn):
            @pl.when(s < _NS - 1)
            def _():
                fn(pl.ds(s * _RPS, _RPS))

            @pl.when(s == _NS - 1)
            def _():
                fn(pl.ds((_NS - 1) * _RPS, _RPS_LAST))

        my_rows(lambda sl: pltpu.sync_copy(zeros_hbm.at[sl], acc_sh.at[sl]))
        plsc.subcore_barrier()

        def drain_scatter(j):
            pltpu.make_async_copy(rows_v[j], acc_sh.at[dst_v[j]],
                                  ssem[j]).wait()

        def run_group(g, b, drain):
            sv = src_v[b * _S:(b + 1) * _S]
            dv = dst_v[b * _S:(b + 1) * _S]
            sm = isem[b * _S:(b + 1) * _S]
            idx = []
            for j in range(_S):
                base = (w + (g * _S + j) * _NW) * _CHUNK
                idx.append((
                    pltpu.async_copy(src_hbm.at[pl.ds(base, _CHUNK)],
                                     sv[j], sm[j]),
                    pltpu.async_copy(dst_hbm.at[pl.ds(base, _CHUNK)],
                                     dv[j], sm[j]),
                ))
            gathers = []
            for j in range(_S):
                idx[j][0].wait()
                idx[j][1].wait()
                drain(j)
                gathers.append(pltpu.async_copy(hs_hbm.at[sv[j]],
                                                rows_v[j], gsem[j]))
            for j in range(_S):
                gathers[j].wait()
                pltpu.async_copy(rows_v[j], acc_sh.at[dv[j]], ssem[j],
                                 add=True)

        @pl.loop(0, _NG // 2)
        def _(i):
            def drain_first(j):
                @pl.when(i > 0)
                def _():
                    drain_scatter(j)
            run_group(2 * i, 0, drain_first)
            run_group(2 * i + 1, 1, drain_scatter)

        run_group(_NG - 1, 0, drain_scatter)
        for j in range(_S):
            drain_scatter(j)

        @pl.when(w < _TAIL)
        def _():
            base = (_NG * _S * _NW + w) * _CHUNK
            pltpu.sync_copy(src_hbm.at[pl.ds(base, _CHUNK)], src_v[0])
            pltpu.sync_copy(dst_hbm.at[pl.ds(base, _CHUNK)], dst_v[0])
            pltpu.async_copy(hs_hbm.at[src_v[0]], rows_v[0], gsem[0]).wait()
            pltpu.sync_copy(rows_v[0], acc_sh.at[dst_v[0]], add=True)

        plsc.subcore_barrier()
        my_rows(lambda sl: pltpu.sync_copy(acc_sh.at[sl], out_hbm.at[c, sl]))

    return deg_kernel, edge_acc_kernel


def _matmul_body(x_ref, w1_ref, h_ref):
    h_ref[...] = jnp.dot(x_ref[...], w1_ref[...],
                         preferred_element_type=jnp.float32)


def _phase2_body(h_ref, dacc_ref, hs_ref, dinv_ref):
    deg = dacc_ref[0, :_N, 0:1] + dacc_ref[1, :_N, 0:1] + 1.0
    dinv = lax.rsqrt(deg)
    hs_ref[...] = h_ref[...] * dinv
    dinv_ref[...] = dinv


def _phase4_body(acc_ref, hs1_ref, dinv_ref, b1_ref, w2_ref, hs2_ref):
    dinv = dinv_ref[...]
    h = (acc_ref[0, :_N] + acc_ref[1, :_N] + hs1_ref[...]) * dinv + b1_ref[...]
    h = jnp.maximum(h, 0.0)
    hs2_ref[...] = jnp.dot(h, w2_ref[...], preferred_element_type=jnp.float32) * dinv


def _phase6_body(acc_ref, hs2_ref, dinv_ref, b2_ref, batch_ref, maskf_ref,
                 wout_ref, bout_ref, out_ref):
    dinv = dinv_ref[...]
    h = (acc_ref[0, :_N] + acc_ref[1, :_N] + hs2_ref[...]) * dinv + b2_ref[...]
    h = jnp.maximum(h, 0.0)
    gids = lax.broadcasted_iota(jnp.int32, (_G, _N), 0)
    in_g = (batch_ref[...] == gids).astype(jnp.float32)
    in_g_m = in_g * maskf_ref[...]
    ent_sum = jnp.dot(in_g_m, h, preferred_element_type=jnp.float32)
    pool_sum = jnp.dot(in_g, h, preferred_element_type=jnp.float32)
    ent_cnt = jnp.sum(in_g_m, axis=1, keepdims=True)
    node_cnt = jnp.sum(in_g, axis=1, keepdims=True)
    doc = jnp.where(ent_cnt < _EPS,
                    pool_sum / jnp.maximum(node_cnt, 1.0),
                    ent_sum / (ent_cnt + _EPS))
    logits = jnp.dot(doc, wout_ref[...], preferred_element_type=jnp.float32)
    logits = logits + bout_ref[...]
    zmax = jnp.max(logits, axis=1, keepdims=True)
    z = logits - zmax
    out_ref[...] = z - jnp.log(jnp.sum(jnp.exp(z), axis=1, keepdims=True))


def kernel(x_fp16, ei, batch, mask, W1, b1, W2, b2, Wout, bout):
    x_f32 = x_fp16.astype(jnp.float32)
    src = ei[0].astype(jnp.int32)
    dst = ei[1].astype(jnp.int32)
    batch_i = batch.astype(jnp.int32).reshape(1, _N)
    maskf = mask.astype(jnp.float32).reshape(1, _N)
    onesH = jnp.ones((_CHUNK, _H), jnp.float32)
    zerosH = jnp.zeros((_N, _H), jnp.float32)

    deg_kernel, edge_acc_kernel = _sc_kernels()
    dacc = deg_kernel(dst, onesH, zerosH)

    h1 = pl.pallas_call(
        _matmul_body,
        out_shape=jax.ShapeDtypeStruct((_N, _H), jnp.float32),
    )(x_f32, W1)

    hs1, dinv = pl.pallas_call(
        _phase2_body,
        out_shape=(
            jax.ShapeDtypeStruct((_N, _H), jnp.float32),
            jax.ShapeDtypeStruct((_N, 1), jnp.float32),
        ),
    )(h1, dacc)

    acc1 = edge_acc_kernel(hs1, src, dst, zerosH)

    hs2 = pl.pallas_call(
        _phase4_body,
        out_shape=jax.ShapeDtypeStruct((_N, _H), jnp.float32),
    )(acc1, hs1, dinv, b1.reshape(1, _H), W2)

    acc2 = edge_acc_kernel(hs2, src, dst, zerosH)

    out = pl.pallas_call(
        _phase6_body,
        out_shape=jax.ShapeDtypeStruct((_G, bout.shape[0]), jnp.float32),
    )(acc2, hs2, dinv, b2.reshape(1, _H), batch_i, maskf, Wout,
      bout.reshape(1, bout.shape[0]))
    return out
